# Initial kernel scaffold; baseline (speedup 1.0000x reference)
import functools
import jax
import jax.numpy as jnp
from jax import lax
from jax.experimental import pallas as pl
from jax.experimental.pallas import tpu as pltpu
from jax.experimental.pallas import tpu_sc as plsc

EMB_DIM = 64
N_NEG = 20
LANES = 16
DCH = EMB_DIM // LANES


def _make_sc_partials(B):
    info = plsc.get_sparse_core_info()
    NC, NS = info.num_cores, info.num_subcores
    NW = NC * NS
    per_w = B // NW
    C = 64
    n_chunks = per_w // C
    NEG_I = C * N_NEG
    NSUB = NEG_I // 128

    mesh = plsc.VectorSubcoreMesh(core_axis_name="c", subcore_axis_name="s")

    @functools.partial(
        pl.kernel,
        mesh=mesh,
        out_type=[
            jax.ShapeDtypeStruct((B, LANES), jnp.float32),
            jax.ShapeDtypeStruct((B, LANES), jnp.float32),
        ],
        scratch_types=[
            pltpu.VMEM((C,), jnp.int32),
            pltpu.VMEM((C,), jnp.int32),
            pltpu.VMEM((NSUB, 128), jnp.int32),
            pltpu.VMEM((C, EMB_DIM), jnp.float32),
            pltpu.VMEM((C, EMB_DIM), jnp.float32),
            pltpu.VMEM((NEG_I, EMB_DIM), jnp.float32),
            pltpu.VMEM((C, LANES), jnp.float32),
            pltpu.VMEM((C, LANES), jnp.float32),
            pltpu.SemaphoreType.DMA,
        ],
    )
    def sc_kernel(tgt_hbm, ctx_hbm, negidx_hbm, u_hbm, v_hbm,
                  pos_hbm, negp_hbm,
                  tgt_v, ctx_v, negidx_v, u_v, v_v, neg_v,
                  posbuf, negbuf, sem):
        wid = lax.axis_index("s") * NC + lax.axis_index("c")
        base_w = wid * per_w

        def chunk_body(ci, carry):
            base = base_w + ci * C
            pltpu.sync_copy(tgt_hbm.at[pl.ds(base, C)], tgt_v)
            pltpu.sync_copy(ctx_hbm.at[pl.ds(base, C)], ctx_v)
            pltpu.sync_copy(
                negidx_hbm.at[pl.ds(base * N_NEG // 128, NSUB)], negidx_v)

            copies = [
                pltpu.async_copy(u_hbm.at[tgt_v], u_v, sem),
                pltpu.async_copy(v_hbm.at[ctx_v], v_v, sem),
            ]
            for j in range(NSUB):
                copies.append(pltpu.async_copy(
                    v_hbm.at[negidx_v.at[j]],
                    neg_v.at[pl.ds(j * 128, 128)], sem))
            for cp in copies:
                cp.wait()

            def elem_body(i, carry2):
                pos = None
                negp = None
                for kk in range(DCH):
                    sl = pl.ds(kk * LANES, LANES)
                    uk = u_v[i, sl]
                    vk = v_v[i, sl]
                    acc = neg_v[i * N_NEG, sl]
                    for n in range(1, N_NEG):
                        acc = acc + neg_v[i * N_NEG + n, sl]
                    pk = uk * vk
                    nk = uk * acc
                    pos = pk if pos is None else pos + pk
                    negp = nk if negp is None else negp + nk
                posbuf[i, :] = pos
                negbuf[i, :] = negp
                return carry2

            lax.fori_loop(0, C, elem_body, 0)
            pltpu.sync_copy(posbuf, pos_hbm.at[pl.ds(base, C)])
            pltpu.sync_copy(negbuf, negp_hbm.at[pl.ds(base, C)])
            return carry

        lax.fori_loop(0, n_chunks, chunk_body, 0)

    return sc_kernel


def _tc_finish(pos_part, neg_part):
    def body(p_ref, n_ref, o_ref):
        p = jnp.sum(p_ref[...], axis=1)
        q = jnp.sum(n_ref[...], axis=1)

        def logsig(x):
            return jnp.minimum(x, 0.0) - jnp.log1p(jnp.exp(-jnp.abs(x)))

        loss = logsig(p) + logsig(-q)
        o_ref[...] = jnp.broadcast_to(-jnp.mean(loss), (1, 1))

    out = pl.pallas_call(
        body,
        out_shape=jax.ShapeDtypeStruct((1, 1), jnp.float32),
    )(pos_part, neg_part)
    return out[0, 0]


def kernel(target_word, context_word, neg_word, u_weight, v_weight):
    B = target_word.shape[0]
    neg2d = neg_word.reshape(B * N_NEG // 128, 128)
    sc = _make_sc_partials(B)
    pos_part, neg_part = sc(target_word, context_word, neg2d,
                            u_weight, v_weight)
    return _tc_finish(pos_part, neg_part)


# speedup vs baseline: 5.2170x; 5.2170x over previous
import functools
import jax
import jax.numpy as jnp
from jax import lax
from jax.experimental import pallas as pl
from jax.experimental.pallas import tpu as pltpu
from jax.experimental.pallas import tpu_sc as plsc

EMB_DIM = 64
N_NEG = 20
LANES = 16
DCH = EMB_DIM // LANES


def _make_sc_partials(B):
    info = plsc.get_sparse_core_info()
    NC, NS = info.num_cores, info.num_subcores
    NW = NC * NS
    per_w = B // NW
    C = 64
    n_chunks = per_w // C
    NEG_I = C * N_NEG
    NSUB = NEG_I // 128

    mesh = plsc.VectorSubcoreMesh(core_axis_name="c", subcore_axis_name="s")

    @functools.partial(
        pl.kernel,
        mesh=mesh,
        compiler_params=pltpu.CompilerParams(use_tc_tiling_on_sc=False),
        out_type=[
            jax.ShapeDtypeStruct((B, LANES), jnp.float32),
            jax.ShapeDtypeStruct((B, LANES), jnp.float32),
        ],
        scratch_types=[
            pltpu.VMEM((C,), jnp.int32),
            pltpu.VMEM((C,), jnp.int32),
            pltpu.VMEM((NEG_I,), jnp.int32),
            pltpu.VMEM((C, EMB_DIM), jnp.float32),
            pltpu.VMEM((C, EMB_DIM), jnp.float32),
            pltpu.VMEM((NEG_I, EMB_DIM), jnp.float32),
            pltpu.VMEM((C, LANES), jnp.float32),
            pltpu.VMEM((C, LANES), jnp.float32),
            pltpu.SemaphoreType.DMA,
        ],
    )
    def sc_kernel(tgt_hbm, ctx_hbm, negidx_hbm, u_hbm, v_hbm,
                  pos_hbm, negp_hbm,
                  tgt_v, ctx_v, negidx_v, u_v, v_v, neg_v,
                  posbuf, negbuf, sem):
        wid = lax.axis_index("s") * NC + lax.axis_index("c")
        base_w = wid * per_w

        def chunk_body(ci, carry):
            base = base_w + ci * C
            pltpu.sync_copy(tgt_hbm.at[pl.ds(base, C)], tgt_v)
            pltpu.sync_copy(ctx_hbm.at[pl.ds(base, C)], ctx_v)
            pltpu.sync_copy(
                negidx_hbm.at[pl.ds(base * N_NEG, NEG_I)], negidx_v)

            copies = [
                pltpu.async_copy(u_hbm.at[tgt_v], u_v, sem),
                pltpu.async_copy(v_hbm.at[ctx_v], v_v, sem),
            ]
            for j in range(NSUB):
                copies.append(pltpu.async_copy(
                    v_hbm.at[negidx_v.at[pl.ds(j * 128, 128)]],
                    neg_v.at[pl.ds(j * 128, 128)], sem))
            for cp in copies:
                cp.wait()

            def elem_body(i, carry2):
                pos = None
                negp = None
                for kk in range(DCH):
                    sl = pl.ds(kk * LANES, LANES)
                    uk = u_v[i, sl]
                    vk = v_v[i, sl]
                    acc = neg_v[i * N_NEG, sl]
                    for n in range(1, N_NEG):
                        acc = acc + neg_v[i * N_NEG + n, sl]
                    pk = uk * vk
                    nk = uk * acc
                    pos = pk if pos is None else pos + pk
                    negp = nk if negp is None else negp + nk
                posbuf[i, :] = pos
                negbuf[i, :] = negp
                return carry2

            lax.fori_loop(0, C, elem_body, 0)
            pltpu.sync_copy(posbuf, pos_hbm.at[pl.ds(base, C)])
            pltpu.sync_copy(negbuf, negp_hbm.at[pl.ds(base, C)])
            return carry

        lax.fori_loop(0, n_chunks, chunk_body, 0)

    return sc_kernel


def _tc_finish(pos_part, neg_part):
    def body(p_ref, n_ref, o_ref):
        p = jnp.sum(p_ref[...], axis=1)
        q = jnp.sum(n_ref[...], axis=1)

        def logsig(x):
            return jnp.minimum(x, 0.0) - jnp.log1p(jnp.exp(-jnp.abs(x)))

        loss = logsig(p) + logsig(-q)
        o_ref[...] = jnp.broadcast_to(-jnp.mean(loss), (1, 1))

    out = pl.pallas_call(
        body,
        out_shape=jax.ShapeDtypeStruct((1, 1), jnp.float32),
    )(pos_part, neg_part)
    return out[0, 0]


def kernel(target_word, context_word, neg_word, u_weight, v_weight):
    B = target_word.shape[0]
    neg_flat = neg_word.reshape(B * N_NEG)
    sc = _make_sc_partials(B)
    pos_part, neg_part = sc(target_word, context_word, neg_flat,
                            u_weight, v_weight)
    return _tc_finish(pos_part, neg_part)
